# baseline (speedup 1.0000x reference)
import functools

import jax
import jax.numpy as jnp
from jax import lax
from jax.experimental import pallas as pl
from jax.experimental.pallas import tpu as pltpu
from jax.experimental.pallas import tpu_sc as plsc

_NUM_CORES = 2
_NUM_SUBCORES = 16
_NW = _NUM_CORES * _NUM_SUBCORES
_LANES = 16


def _emb_kernel(B, S, D, V):
    b_per_w = B // _NW
    s_half = S // 2
    half = b_per_w // 2
    mesh = plsc.VectorSubcoreMesh(
        core_axis_name="c", subcore_axis_name="s",
        num_cores=_NUM_CORES, num_subcores=_NUM_SUBCORES)

    @functools.partial(
        pl.kernel,
        out_type=jax.ShapeDtypeStruct((B, S, D), jnp.float32),
        mesh=mesh,
        scratch_types=[
            pltpu.VMEM((b_per_w, 2, s_half), jnp.int32),
            pltpu.VMEM((S, D), jnp.float32),
            pltpu.VMEM((S, D), jnp.float32),
            pltpu.VMEM((S, D), jnp.float32),
            pltpu.SemaphoreType.DMA,
            pltpu.SemaphoreType.DMA,
            pltpu.SemaphoreType.DMA,
            pltpu.SemaphoreType.DMA,
        ],
    )
    def body(x_hbm, pos_hbm, tok_hbm, out_hbm,
             idx_v, pos_v, buf0, buf1, g0, g1, o0, o1):
        wid = lax.axis_index("s") * _NUM_CORES + lax.axis_index("c")
        base = wid * b_per_w
        bufs, gsems, osems = (buf0, buf1), (g0, g1), (o0, o1)

        pltpu.sync_copy(x_hbm.at[pl.ds(base, b_per_w)], idx_v)
        pltpu.sync_copy(pos_hbm, pos_v)

        def gather(i, buf, sem):
            for j in range(2):
                pltpu.async_copy(
                    tok_hbm.at[idx_v.at[i, j]],
                    buf.at[pl.ds(j * s_half, s_half)], sem)

        def gather_wait(buf, sem):
            for j in range(2):
                pltpu.make_async_copy(
                    tok_hbm.at[idx_v.at[0, j]],
                    buf.at[pl.ds(j * s_half, s_half)], sem).wait()

        def out_wait(buf, sem):
            pltpu.make_async_copy(buf, out_hbm.at[base], sem).wait()

        def add_pos(buf):
            def add_row(r, c):
                for ch in range(D // _LANES):
                    sl = pl.ds(ch * _LANES, _LANES)
                    plsc.addupdate(buf.at[r, sl], pos_v[r, sl])
                return c
            lax.fori_loop(0, S, add_row, 0, unroll=2)

        gather(0, buf0, g0)

        def step(g, carry):
            for k in range(2):
                i = 2 * g + k
                gather_wait(bufs[k], gsems[k])
                if k == 0:
                    @pl.when(g >= 1)
                    def _():
                        out_wait(bufs[1], osems[1])
                    gather(i + 1, bufs[1], gsems[1])
                else:
                    out_wait(bufs[0], osems[0])
                    @pl.when(g < half - 1)
                    def _():
                        gather(i + 1, bufs[0], gsems[0])
                add_pos(bufs[k])
                pltpu.async_copy(bufs[k], out_hbm.at[base + i], osems[k])
            return carry

        lax.fori_loop(0, half, step, 0, unroll=False)
        out_wait(buf1, o1)

    return body


def kernel(x, pos_table, token_table):
    B, S = x.shape
    V, D = token_table.shape
    x3 = x.astype(jnp.int32).reshape(B, 2, S // 2)
    out = _emb_kernel(B, S, D, V)(x3, pos_table, token_table)
    return out


# speedup vs baseline: 1.1455x; 1.1455x over previous
import functools

import jax
import jax.numpy as jnp
from jax import lax
from jax.experimental import pallas as pl
from jax.experimental.pallas import tpu as pltpu
from jax.experimental.pallas import tpu_sc as plsc

_NUM_CORES = 2
_NUM_SUBCORES = 16
_NW = _NUM_CORES * _NUM_SUBCORES
_LANES = 16
_NBUF = 4
_SA = 104
_SB = 96


def _emb_kernel(B, S, D, V):
    b_per_w = B // _NW
    n_slots = 2 * b_per_w
    trips = n_slots // _NBUF
    mesh = plsc.VectorSubcoreMesh(
        core_axis_name="c", subcore_axis_name="s",
        num_cores=_NUM_CORES, num_subcores=_NUM_SUBCORES)
    slot_rows = (_SA, _SB)

    @functools.partial(
        pl.kernel,
        out_type=jax.ShapeDtypeStruct((B, S, D), jnp.float32),
        mesh=mesh,
        scratch_types=[
            pltpu.VMEM((b_per_w, _SA), jnp.int32),
            pltpu.VMEM((b_per_w, _SB), jnp.int32),
            pltpu.VMEM((S, D), jnp.float32),
            [pltpu.VMEM((_SA, D), jnp.float32) for _ in range(_NBUF)],
            [pltpu.SemaphoreType.DMA for _ in range(_NBUF)],
            [pltpu.SemaphoreType.DMA for _ in range(_NBUF)],
        ],
    )
    def body(xa_hbm, xb_hbm, pos_hbm, tok_hbm, out_hbm,
             idx_a, idx_b, pos_v, bufs, gs, os):
        wid = lax.axis_index("s") * _NUM_CORES + lax.axis_index("c")
        base = wid * b_per_w
        idxs = (idx_a, idx_b)

        pltpu.sync_copy(xa_hbm.at[pl.ds(base, b_per_w)], idx_a)
        pltpu.sync_copy(xb_hbm.at[pl.ds(base, b_per_w)], idx_b)
        pltpu.sync_copy(pos_hbm, pos_v)

        def gather(bat, k):
            h = k % 2
            pltpu.async_copy(tok_hbm.at[idxs[h].at[bat]],
                             bufs[k].at[pl.ds(0, slot_rows[h])], gs[k])

        def gather_wait(k):
            h = k % 2
            pltpu.make_async_copy(tok_hbm.at[idxs[h].at[0]],
                                  bufs[k].at[pl.ds(0, slot_rows[h])],
                                  gs[k]).wait()

        def out_issue(bat, k):
            h = k % 2
            pltpu.async_copy(
                bufs[k].at[pl.ds(0, slot_rows[h])],
                out_hbm.at[base + bat, pl.ds(h * _SA, slot_rows[h])], os[k])

        def out_wait(k):
            h = k % 2
            pltpu.make_async_copy(
                bufs[k].at[pl.ds(0, slot_rows[h])],
                out_hbm.at[base, pl.ds(h * _SA, slot_rows[h])], os[k]).wait()

        def add_pos(k):
            h = k % 2

            def add_row(r, c):
                for ch in range(D // _LANES):
                    sl = pl.ds(ch * _LANES, _LANES)
                    plsc.addupdate(bufs[k].at[r, sl],
                                   pos_v[h * _SA + r, sl])
                return c
            lax.fori_loop(0, slot_rows[h], add_row, 0, unroll=2)

        gather(0, 0)
        gather(0, 1)

        def step(g, carry):
            for k in range(_NBUF):
                bat = 2 * g + k // 2
                gather_wait(k)
                kn = (k + 2) % _NBUF
                if k < 2:
                    @pl.when(g >= 1)
                    def _():
                        out_wait(kn)
                    gather(bat + 1, kn)
                else:
                    out_wait(kn)
                    @pl.when(g < trips - 1)
                    def _():
                        gather(bat + 1, kn)
                add_pos(k)
                out_issue(bat, k)
            return carry

        lax.fori_loop(0, trips, step, 0, unroll=False)
        out_wait(2)
        out_wait(3)

    return body


def kernel(x, pos_table, token_table):
    B, S = x.shape
    V, D = token_table.shape
    xi = x.astype(jnp.int32)
    out = _emb_kernel(B, S, D, V)(
        xi[:, :_SA], xi[:, _SA:], pos_table, token_table)
    return out
